# baseline (speedup 1.0000x reference)
import functools

import jax
import jax.numpy as jnp
from jax import lax
from jax.experimental import pallas as pl
from jax.experimental.pallas import tpu as pltpu
from jax.experimental.pallas import tpu_sc as plsc

DIM = 64
NVREG = DIM // 16
CHUNK = 100


def _make_pool(B, L):
    info = plsc.get_sparse_core_info()
    NC, NS = info.num_cores, info.num_subcores
    NW = NC * NS
    assert B % NW == 0 and L % CHUNK == 0
    chunks_per_seq = L // CHUNK
    seq_per_w = B // NW
    rows_per_w = seq_per_w * chunks_per_seq
    mesh = plsc.VectorSubcoreMesh(core_axis_name="c", subcore_axis_name="s")

    @functools.partial(
        pl.kernel,
        mesh=mesh,
        compiler_params=pltpu.CompilerParams(use_tc_tiling_on_sc=False),
        out_type=jax.ShapeDtypeStruct((B, DIM), jnp.float32),
        scratch_types=[
            pltpu.VMEM((rows_per_w, CHUNK), jnp.int32),
            pltpu.VMEM((CHUNK, DIM), jnp.float32),
            pltpu.VMEM((CHUNK, DIM), jnp.float32),
            pltpu.VMEM((seq_per_w, DIM), jnp.float32),
            pltpu.SemaphoreType.DMA,
            pltpu.SemaphoreType.DMA,
        ],
    )
    def pool(idx_hbm, table_hbm, out_hbm, idx_v, buf0, buf1, pooled_v, sem0, sem1):
        wid = lax.axis_index("s") * NC + lax.axis_index("c")
        bufs = (buf0, buf1)
        sems = (sem0, sem1)

        pltpu.sync_copy(idx_hbm.at[pl.ds(wid * rows_per_w, rows_per_w)], idx_v)

        def gather(c, i):
            return pltpu.make_async_copy(
                table_hbm.at[idx_v.at[c]], bufs[i], sems[i])

        gather(0, 0).start()

        def accum(buf, a):
            def body(j, a):
                return tuple(
                    a[k] + buf[j, pl.ds(16 * k, 16)] for k in range(NVREG))
            return lax.fori_loop(0, CHUNK, body, a, unroll=4)

        zero = jnp.zeros((16,), jnp.float32)

        def seq_body(s, carry):
            a = (zero,) * NVREG
            for i in range(chunks_per_seq):
                c = chunks_per_seq * s + i
                gather(c, i % 2).wait()

                @pl.when(c + 1 < rows_per_w)
                def _start_next():
                    gather(c + 1, (i + 1) % 2).start()

                a = accum(bufs[i % 2], a)
            for k in range(NVREG):
                pooled_v[s, pl.ds(16 * k, 16)] = a[k]
            return carry

        lax.fori_loop(0, seq_per_w, seq_body, 0)
        pltpu.sync_copy(pooled_v, out_hbm.at[pl.ds(wid * seq_per_w, seq_per_w)])

    return pool


def _linear(pooled, W, b):
    def body(x_ref, w_ref, b_ref, o_ref):
        o_ref[...] = lax.dot_general(
            x_ref[...], w_ref[...],
            (((1,), (1,)), ((), ())),
            preferred_element_type=jnp.float32,
        ) + b_ref[...]

    return pl.pallas_call(
        body,
        out_shape=jax.ShapeDtypeStruct((pooled.shape[0], W.shape[0]), jnp.float32),
    )(pooled, W, b.reshape(1, -1))


def kernel(tinputs, tinputs_len, table, W, b):
    B, L = tinputs.shape
    idx = tinputs.astype(jnp.int32).reshape(B * L // CHUNK, CHUNK)
    pooled = _make_pool(B, L)(idx, table)
    return _linear(pooled, W, b)


# speedup vs baseline: 1.1932x; 1.1932x over previous
import functools

import jax
import jax.numpy as jnp
from jax import lax
from jax.experimental import pallas as pl
from jax.experimental.pallas import tpu as pltpu
from jax.experimental.pallas import tpu_sc as plsc

DIM = 64
NVREG = DIM // 16
CHUNK = 100


def _make_pool(B, L):
    info = plsc.get_sparse_core_info()
    NC, NS = info.num_cores, info.num_subcores
    NW = NC * NS
    assert B % NW == 0 and L % CHUNK == 0
    chunks_per_seq = L // CHUNK
    seq_per_w = B // NW
    rows_per_w = seq_per_w * chunks_per_seq
    mesh = plsc.VectorSubcoreMesh(core_axis_name="c", subcore_axis_name="s")

    @functools.partial(
        pl.kernel,
        mesh=mesh,
        compiler_params=pltpu.CompilerParams(use_tc_tiling_on_sc=False),
        out_type=jax.ShapeDtypeStruct((B, DIM), jnp.float32),
        scratch_types=[
            pltpu.VMEM((rows_per_w, CHUNK), jnp.int32),
            pltpu.VMEM((CHUNK, DIM), jnp.float32),
            pltpu.VMEM((CHUNK, DIM), jnp.float32),
            pltpu.VMEM((CHUNK, DIM), jnp.float32),
            pltpu.VMEM((CHUNK, DIM), jnp.float32),
            pltpu.VMEM((seq_per_w, DIM), jnp.float32),
            pltpu.SemaphoreType.DMA,
            pltpu.SemaphoreType.DMA,
            pltpu.SemaphoreType.DMA,
            pltpu.SemaphoreType.DMA,
        ],
    )
    def pool(idx_hbm, table_hbm, out_hbm, idx_v,
             buf0, buf1, buf2, buf3, pooled_v, sem0, sem1, sem2, sem3):
        wid = lax.axis_index("s") * NC + lax.axis_index("c")
        bufs = (buf0, buf1, buf2, buf3)
        sems = (sem0, sem1, sem2, sem3)
        nbuf = len(bufs)

        pltpu.sync_copy(idx_hbm.at[pl.ds(wid * rows_per_w, rows_per_w)], idx_v)

        def gather(c, i):
            return pltpu.make_async_copy(
                table_hbm.at[idx_v.at[c]], bufs[i], sems[i])

        for c in range(nbuf - 1):
            gather(c, c).start()

        def accum(buf, a):
            def body(j, a):
                return tuple(
                    a[k] + buf[j, pl.ds(16 * k, 16)] for k in range(NVREG))
            return lax.fori_loop(0, CHUNK, body, a, unroll=4)

        zero = jnp.zeros((16,), jnp.float32)

        def outer_body(ss, carry):
            for so in range(nbuf // chunks_per_seq):
                s = (nbuf // chunks_per_seq) * ss + so
                a = (zero,) * NVREG
                for i in range(chunks_per_seq):
                    u = chunks_per_seq * so + i
                    c = nbuf * ss + u
                    gather(c, u).wait()

                    @pl.when(c + nbuf - 1 < rows_per_w)
                    def _start_next():
                        gather(c + nbuf - 1, (u + nbuf - 1) % nbuf).start()

                    a = accum(bufs[u], a)
                for k in range(NVREG):
                    pooled_v[s, pl.ds(16 * k, 16)] = a[k]
            return carry

        lax.fori_loop(0, rows_per_w // nbuf, outer_body, 0)
        pltpu.sync_copy(pooled_v, out_hbm.at[pl.ds(wid * seq_per_w, seq_per_w)])

    return pool


def _linear(pooled, W, b):
    def body(x_ref, w_ref, b_ref, o_ref):
        o_ref[...] = lax.dot_general(
            x_ref[...], w_ref[...],
            (((1,), (1,)), ((), ())),
            preferred_element_type=jnp.float32,
        ) + b_ref[...]

    return pl.pallas_call(
        body,
        out_shape=jax.ShapeDtypeStruct((pooled.shape[0], W.shape[0]), jnp.float32),
    )(pooled, W, b.reshape(1, -1))


def kernel(tinputs, tinputs_len, table, W, b):
    B, L = tinputs.shape
    idx = tinputs.astype(jnp.int32).reshape(B * L // CHUNK, CHUNK)
    pooled = _make_pool(B, L)(idx, table)
    return _linear(pooled, W, b)


# speedup vs baseline: 1.2279x; 1.0291x over previous
import functools

import jax
import jax.numpy as jnp
from jax import lax
from jax.experimental import pallas as pl
from jax.experimental.pallas import tpu as pltpu
from jax.experimental.pallas import tpu_sc as plsc

DIM = 64
NVREG = DIM // 16
CHUNK = 100
NBUF = 8


def _make_pool(B, L):
    info = plsc.get_sparse_core_info()
    NC, NS = info.num_cores, info.num_subcores
    NW = NC * NS
    assert B % NW == 0 and L % CHUNK == 0
    chunks_per_seq = L // CHUNK
    seq_per_w = B // NW
    rows_per_w = seq_per_w * chunks_per_seq
    mesh = plsc.VectorSubcoreMesh(core_axis_name="c", subcore_axis_name="s")

    @functools.partial(
        pl.kernel,
        mesh=mesh,
        compiler_params=pltpu.CompilerParams(use_tc_tiling_on_sc=False),
        out_type=jax.ShapeDtypeStruct((B, DIM), jnp.float32),
        scratch_types=[
            pltpu.VMEM((rows_per_w, CHUNK), jnp.int32),
            *([pltpu.VMEM((CHUNK, DIM), jnp.float32)] * NBUF),
            pltpu.VMEM((seq_per_w, DIM), jnp.float32),
            *([pltpu.SemaphoreType.DMA] * NBUF),
        ],
    )
    def pool(idx_hbm, table_hbm, out_hbm, idx_v, *rest):
        bufs = rest[:NBUF]
        pooled_v = rest[NBUF]
        sems = rest[NBUF + 1:]
        nbuf = NBUF
        wid = lax.axis_index("s") * NC + lax.axis_index("c")

        pltpu.sync_copy(idx_hbm.at[pl.ds(wid * rows_per_w, rows_per_w)], idx_v)

        def gather(c, i):
            return pltpu.make_async_copy(
                table_hbm.at[idx_v.at[c]], bufs[i], sems[i])

        for c in range(nbuf - 1):
            gather(c, c).start()

        def accum(buf, a):
            def body(j, a):
                return tuple(
                    a[k] + buf[j, pl.ds(16 * k, 16)] for k in range(NVREG))
            return lax.fori_loop(0, CHUNK, body, a, unroll=4)

        zero = jnp.zeros((16,), jnp.float32)

        def outer_body(ss, carry):
            for so in range(nbuf // chunks_per_seq):
                s = (nbuf // chunks_per_seq) * ss + so
                a = (zero,) * NVREG
                for i in range(chunks_per_seq):
                    u = chunks_per_seq * so + i
                    c = nbuf * ss + u
                    gather(c, u).wait()

                    @pl.when(c + nbuf - 1 < rows_per_w)
                    def _start_next():
                        gather(c + nbuf - 1, (u + nbuf - 1) % nbuf).start()

                    a = accum(bufs[u], a)
                for k in range(NVREG):
                    pooled_v[s, pl.ds(16 * k, 16)] = a[k]
            return carry

        lax.fori_loop(0, rows_per_w // nbuf, outer_body, 0)
        pltpu.sync_copy(pooled_v, out_hbm.at[pl.ds(wid * seq_per_w, seq_per_w)])

    return pool


def _linear(pooled, W, b):
    def body(x_ref, w_ref, b_ref, o_ref):
        o_ref[...] = lax.dot_general(
            x_ref[...], w_ref[...],
            (((1,), (1,)), ((), ())),
            preferred_element_type=jnp.float32,
        ) + b_ref[...]

    return pl.pallas_call(
        body,
        out_shape=jax.ShapeDtypeStruct((pooled.shape[0], W.shape[0]), jnp.float32),
    )(pooled, W, b.reshape(1, -1))


def kernel(tinputs, tinputs_len, table, W, b):
    B, L = tinputs.shape
    idx = tinputs.astype(jnp.int32).reshape(B * L // CHUNK, CHUNK)
    pooled = _make_pool(B, L)(idx, table)
    return _linear(pooled, W, b)
